# baseline (speedup 1.0000x reference)
import functools

import jax
import jax.numpy as jnp
from jax import lax
from jax.experimental import pallas as pl
from jax.experimental.pallas import tpu as pltpu
from jax.experimental.pallas import tpu_sc as plsc

N = 10000
N_PAD = 10112
ROWS_PER_SUB = N_PAD // 16
F_IN = 128
H = 128
Z = 64

E = 160000
NW = 32
EPT = 5120
E_PAD = EPT * NW
K = 128
NCHUNK = EPT // K

BM = 512

_MESH = plsc.VectorSubcoreMesh(core_axis_name="c", subcore_axis_name="s")


def _sc_scatter_body(width, u_hbm, zeros_hbm, src_hbm, dst_hbm, out_hbm,
                     acc, sidx, didx, rows0, rows1, gsem0, gsem1):
    c = lax.axis_index("c")
    s = lax.axis_index("s")
    wid = s * 2 + c

    pltpu.sync_copy(src_hbm.at[wid], sidx)
    pltpu.sync_copy(dst_hbm.at[wid], didx)

    r0 = s * ROWS_PER_SUB

    @pl.when(c == 0)
    def _():
        pltpu.sync_copy(u_hbm.at[pl.ds(r0, ROWS_PER_SUB)],
                        acc.at[pl.ds(r0, ROWS_PER_SUB)])

    @pl.when(c != 0)
    def _():
        pltpu.sync_copy(zeros_hbm.at[pl.ds(r0, ROWS_PER_SUB)],
                        acc.at[pl.ds(r0, ROWS_PER_SUB)])

    plsc.subcore_barrier()

    def gather(j, rows, sem):
        pltpu.async_copy(u_hbm.at[sidx.at[j]], rows, sem)

    def wait_gather(rows, sem):
        pltpu.make_async_copy(u_hbm.at[sidx.at[0]], rows, sem).wait()

    gather(0, rows0, gsem0)
    gather(1, rows1, gsem1)

    def body(i, carry):
        j = 2 * i
        wait_gather(rows0, gsem0)
        pltpu.sync_copy(rows0, acc.at[didx.at[j]], add=True)

        @pl.when(j + 2 < NCHUNK)
        def _():
            gather(j + 2, rows0, gsem0)

        wait_gather(rows1, gsem1)
        pltpu.sync_copy(rows1, acc.at[didx.at[j + 1]], add=True)

        @pl.when(j + 3 < NCHUNK)
        def _():
            gather(j + 3, rows1, gsem1)

        return carry

    lax.fori_loop(0, NCHUNK // 2, body, 0)
    plsc.subcore_barrier()

    pltpu.sync_copy(acc.at[pl.ds(r0, ROWS_PER_SUB)],
                    out_hbm.at[c, pl.ds(r0, ROWS_PER_SUB)])


def _make_sc_scatter(width):
    return pl.kernel(
        functools.partial(_sc_scatter_body, width),
        mesh=_MESH,
        out_type=jax.ShapeDtypeStruct((2, N_PAD, width), jnp.float32),
        scratch_types=[
            pltpu.VMEM_SHARED((N_PAD, width), jnp.float32),
            pltpu.VMEM((NCHUNK, K), jnp.int32),
            pltpu.VMEM((NCHUNK, K), jnp.int32),
            pltpu.VMEM((K, width), jnp.float32),
            pltpu.VMEM((K, width), jnp.float32),
            pltpu.SemaphoreType.DMA,
            pltpu.SemaphoreType.DMA,
        ],
    )


_sc_scatter_128 = _make_sc_scatter(128)


def _sc_deg_body(ones_hbm, zeros_hbm, dst_hbm, out_hbm, acc, didx, ones_v, sem):
    c = lax.axis_index("c")
    s = lax.axis_index("s")
    wid = s * 2 + c
    r0 = s * ROWS_PER_SUB

    pltpu.sync_copy(dst_hbm.at[wid], didx)
    pltpu.sync_copy(zeros_hbm.at[pl.ds(r0, ROWS_PER_SUB)],
                    acc.at[pl.ds(r0, ROWS_PER_SUB)])
    pltpu.sync_copy(ones_hbm, ones_v)
    plsc.subcore_barrier()

    def body(j, carry):
        pltpu.sync_copy(ones_v, acc.at[didx.at[j]], add=True)
        return carry

    lax.fori_loop(0, NCHUNK, body, 0)
    plsc.subcore_barrier()

    pltpu.sync_copy(acc.at[pl.ds(r0, ROWS_PER_SUB)],
                    out_hbm.at[c, pl.ds(r0, ROWS_PER_SUB)])


_sc_deg = pl.kernel(
    _sc_deg_body,
    mesh=_MESH,
    out_type=jax.ShapeDtypeStruct((2, N_PAD, 128), jnp.float32),
    scratch_types=[
        pltpu.VMEM_SHARED((N_PAD, 128), jnp.float32),
        pltpu.VMEM((NCHUNK, K), jnp.int32),
        pltpu.VMEM((K, 128), jnp.float32),
        pltpu.SemaphoreType.DMA,
    ],
)


def _prep_kernel(dega_ref, degb_ref, x_ref, w_ref, dinv_ref, u_ref):
    deg = dega_ref[...][:, :1] + degb_ref[...][:, :1] + 1.0
    dinv = lax.rsqrt(deg)
    dinv_ref[...] = dinv
    u_ref[...] = dinv * jnp.dot(x_ref[...], w_ref[...],
                                preferred_element_type=jnp.float32)


def _prep(dega, degb, x, w):
    g = pl.cdiv(N_PAD, BM)
    return pl.pallas_call(
        _prep_kernel,
        grid=(g,),
        in_specs=[
            pl.BlockSpec((BM, 128), lambda i: (i, 0)),
            pl.BlockSpec((BM, 128), lambda i: (i, 0)),
            pl.BlockSpec((BM, F_IN), lambda i: (i, 0)),
            pl.BlockSpec((F_IN, H), lambda i: (0, 0)),
        ],
        out_specs=[
            pl.BlockSpec((BM, 1), lambda i: (i, 0)),
            pl.BlockSpec((BM, H), lambda i: (i, 0)),
        ],
        out_shape=[
            jax.ShapeDtypeStruct((N_PAD, 1), jnp.float32),
            jax.ShapeDtypeStruct((N_PAD, H), jnp.float32),
        ],
    )(dega, degb, x, w)


def _fused_layer_kernel(s_ref, dinv_ref, b_ref, w_ref, o_ref):
    h = jnp.maximum(dinv_ref[...] * (s_ref[0] + s_ref[1]) + b_ref[...], 0.0)
    o_ref[...] = dinv_ref[...] * jnp.dot(h, w_ref[...],
                                         preferred_element_type=jnp.float32)


def _fused_layer(s, dinv, b, w):
    g = pl.cdiv(N_PAD, BM)
    win = s.shape[2]
    return pl.pallas_call(
        _fused_layer_kernel,
        grid=(g,),
        in_specs=[
            pl.BlockSpec((2, BM, win), lambda i: (0, i, 0)),
            pl.BlockSpec((BM, 1), lambda i: (i, 0)),
            pl.BlockSpec((1, win), lambda i: (0, 0)),
            pl.BlockSpec(w.shape, lambda i: (0, 0)),
        ],
        out_specs=pl.BlockSpec((BM, w.shape[1]), lambda i: (i, 0)),
        out_shape=jax.ShapeDtypeStruct((N_PAD, w.shape[1]), jnp.float32),
    )(s, dinv, b, w)


def _mu_kernel(s_ref, dinv_ref, b_ref, o_ref):
    o_ref[...] = dinv_ref[...] * (s_ref[0][:, :Z] + s_ref[1][:, :Z]) + b_ref[...]


def _mu_combine(s, dinv, b):
    g = pl.cdiv(N_PAD, BM)
    return pl.pallas_call(
        _mu_kernel,
        grid=(g,),
        in_specs=[
            pl.BlockSpec((2, BM, 128), lambda i: (0, i, 0)),
            pl.BlockSpec((BM, 1), lambda i: (i, 0)),
            pl.BlockSpec((1, Z), lambda i: (0, 0)),
        ],
        out_specs=pl.BlockSpec((BM, Z), lambda i: (i, 0)),
        out_shape=jax.ShapeDtypeStruct((N_PAD, Z), jnp.float32),
    )(s, dinv, b)


BDM = 1024
BN = 2048


def _decoder_kernel(a_ref, b_ref, o_ref):
    p = lax.dot_general(a_ref[...], b_ref[...], (((1,), (1,)), ((), ())),
                        preferred_element_type=jnp.float32)
    o_ref[...] = 0.5 * jnp.tanh(0.5 * p) + 0.5


def _decoder(z):
    return pl.pallas_call(
        _decoder_kernel,
        grid=(pl.cdiv(N, BDM), pl.cdiv(N, BN)),
        in_specs=[
            pl.BlockSpec((BDM, Z), lambda i, j: (i, 0)),
            pl.BlockSpec((BN, Z), lambda i, j: (j, 0)),
        ],
        out_specs=pl.BlockSpec((BDM, BN), lambda i, j: (i, j)),
        out_shape=jax.ShapeDtypeStruct((N, N), jnp.float32),
    )(z, z)


def kernel(x, edge_index, W1, b1, W2, b2, Wmu, bmu, Wls, bls):
    del Wls, bls
    src = edge_index[0].astype(jnp.int32)
    dst = edge_index[1].astype(jnp.int32)

    npad = E_PAD - E
    pad_idx = (N + (jnp.arange(npad, dtype=jnp.int32) % 16))
    src_p = jnp.concatenate([src, pad_idx]).reshape(NW, NCHUNK, K)
    dst_p = jnp.concatenate([dst, pad_idx]).reshape(NW, NCHUNK, K)

    x_pad = jnp.pad(x, ((0, N_PAD - N), (0, 0)))
    zeros128 = jnp.zeros((N_PAD, 128), jnp.float32)
    ones128 = jnp.ones((K, 128), jnp.float32)
    Wmu_p = jnp.pad(Wmu, ((0, 0), (0, 128 - Z)))

    deg = _sc_deg(ones128, zeros128, dst_p)
    dinv, u1 = _prep(deg[0], deg[1], x_pad, W1)

    s1 = _sc_scatter_128(u1, zeros128, src_p, dst_p)
    u2 = _fused_layer(s1, dinv, b1[None, :], W2)
    s2 = _sc_scatter_128(u2, zeros128, src_p, dst_p)
    u3 = _fused_layer(s2, dinv, b2[None, :], Wmu_p)
    s3 = _sc_scatter_128(u3, zeros128, src_p, dst_p)
    mu = _mu_combine(s3, dinv, bmu[None, :])
    return _decoder(mu)


# speedup vs baseline: 1.0604x; 1.0604x over previous
import functools

import jax
import jax.numpy as jnp
from jax import lax
from jax.experimental import pallas as pl
from jax.experimental.pallas import tpu as pltpu
from jax.experimental.pallas import tpu_sc as plsc

N = 10000
N_PAD = 10112
ROWS_PER_SUB = N_PAD // 16
F_IN = 128
H = 128
Z = 64

E = 160000
NW = 32
K = 64
NCHUNK = 81
EPT = NCHUNK * K
E_PAD = EPT * NW

BM = 2048

_MESH = plsc.VectorSubcoreMesh(core_axis_name="c", subcore_axis_name="s")


NBUF = 3


def _sc_scatter_body(width, u_hbm, zeros_hbm, src_hbm, dst_hbm, out_hbm,
                     acc, sidx, didx, rows, isem, nsem, gsems, ssems):
    c = lax.axis_index("c")
    s = lax.axis_index("s")
    wid = s * 2 + c
    r0 = s * ROWS_PER_SUB

    pltpu.async_copy(src_hbm.at[wid], sidx, isem)
    pltpu.async_copy(dst_hbm.at[wid], didx, isem)

    @pl.when(c == 0)
    def _():
        pltpu.async_copy(u_hbm.at[pl.ds(r0, ROWS_PER_SUB)],
                         acc.at[pl.ds(r0, ROWS_PER_SUB)], nsem)

    @pl.when(c != 0)
    def _():
        pltpu.async_copy(zeros_hbm.at[pl.ds(r0, ROWS_PER_SUB)],
                         acc.at[pl.ds(r0, ROWS_PER_SUB)], nsem)

    pltpu.make_async_copy(src_hbm.at[wid], sidx, isem).wait()
    pltpu.make_async_copy(dst_hbm.at[wid], didx, isem).wait()

    def gather(j, b):
        pltpu.async_copy(u_hbm.at[sidx.at[j]], rows[b], gsems[b])

    def wait_gather(b):
        pltpu.make_async_copy(u_hbm.at[sidx.at[0]], rows[b], gsems[b]).wait()

    def scatter(j, b):
        pltpu.async_copy(rows[b], acc.at[didx.at[j]], ssems[b], add=True)

    def wait_scatter(b):
        pltpu.make_async_copy(rows[b], acc.at[didx.at[0]], ssems[b]).wait()

    gather(0, 0)
    gather(1, 1)
    pltpu.make_async_copy(u_hbm.at[pl.ds(0, ROWS_PER_SUB)],
                          acc.at[pl.ds(0, ROWS_PER_SUB)], nsem).wait()
    plsc.subcore_barrier()

    def body(i, carry):
        for u in range(NBUF):
            j = NBUF * i + u
            b = u
            bp = (u + 2) % NBUF
            wait_gather(b)
            scatter(j, b)
            if u == 0:
                @pl.when(i > 0)
                def _():
                    wait_scatter(bp)
            else:
                wait_scatter(bp)

            @pl.when(j + 2 < NCHUNK)
            def _():
                gather(j + 2, bp)

        return carry

    lax.fori_loop(0, NCHUNK // NBUF, body, 0)
    wait_scatter((NCHUNK - 1) % NBUF)
    plsc.subcore_barrier()

    pltpu.sync_copy(acc.at[pl.ds(r0, ROWS_PER_SUB)],
                    out_hbm.at[c, pl.ds(r0, ROWS_PER_SUB)])


def _make_sc_scatter(width):
    return pl.kernel(
        functools.partial(_sc_scatter_body, width),
        mesh=_MESH,
        out_type=jax.ShapeDtypeStruct((2, N_PAD, width), jnp.float32),
        scratch_types=[
            pltpu.VMEM_SHARED((N_PAD, width), jnp.float32),
            pltpu.VMEM((NCHUNK, K), jnp.int32),
            pltpu.VMEM((NCHUNK, K), jnp.int32),
            [pltpu.VMEM((K, width), jnp.float32)] * NBUF,
            pltpu.SemaphoreType.DMA,
            pltpu.SemaphoreType.DMA,
            [pltpu.SemaphoreType.DMA] * NBUF,
            [pltpu.SemaphoreType.DMA] * NBUF,
        ],
    )


_sc_scatter_128 = _make_sc_scatter(128)


def _sc_deg_body(ones_hbm, zeros_hbm, dst_hbm, out_hbm, acc, didx, ones_v, sem):
    c = lax.axis_index("c")
    s = lax.axis_index("s")
    wid = s * 2 + c
    r0 = s * ROWS_PER_SUB

    pltpu.sync_copy(dst_hbm.at[wid], didx)
    pltpu.sync_copy(zeros_hbm.at[pl.ds(r0, ROWS_PER_SUB)],
                    acc.at[pl.ds(r0, ROWS_PER_SUB)])
    pltpu.sync_copy(ones_hbm, ones_v)
    plsc.subcore_barrier()

    def body(j, carry):
        pltpu.async_copy(ones_v, acc.at[didx.at[j]], sem, add=True)
        return carry

    lax.fori_loop(0, NCHUNK, body, 0)

    def drain(j, carry):
        pltpu.make_async_copy(ones_v, acc.at[didx.at[0]], sem).wait()
        return carry

    lax.fori_loop(0, NCHUNK, drain, 0)
    plsc.subcore_barrier()

    pltpu.sync_copy(acc.at[pl.ds(r0, ROWS_PER_SUB)],
                    out_hbm.at[c, pl.ds(r0, ROWS_PER_SUB)])


_sc_deg = pl.kernel(
    _sc_deg_body,
    mesh=_MESH,
    out_type=jax.ShapeDtypeStruct((2, N_PAD, 128), jnp.float32),
    scratch_types=[
        pltpu.VMEM_SHARED((N_PAD, 128), jnp.float32),
        pltpu.VMEM((NCHUNK, K), jnp.int32),
        pltpu.VMEM((K, 128), jnp.float32),
        pltpu.SemaphoreType.DMA,
    ],
)


def _xw_kernel(x_ref, w_ref, o_ref):
    o_ref[...] = jnp.dot(x_ref[...], w_ref[...],
                         preferred_element_type=jnp.float32)


def _xw(x, w):
    g = pl.cdiv(N_PAD, BM)
    return pl.pallas_call(
        _xw_kernel,
        grid=(g,),
        in_specs=[
            pl.BlockSpec((BM, F_IN), lambda i: (i, 0)),
            pl.BlockSpec((F_IN, H), lambda i: (0, 0)),
        ],
        out_specs=pl.BlockSpec((BM, H), lambda i: (i, 0)),
        out_shape=jax.ShapeDtypeStruct((N_PAD, H), jnp.float32),
    )(x, w)


def _prep_kernel(dega_ref, degb_ref, xw_ref, dinv_ref, u_ref):
    deg = dega_ref[...][:, :1] + degb_ref[...][:, :1] + 1.0
    dinv = lax.rsqrt(deg)
    dinv_ref[...] = dinv
    u_ref[...] = dinv * xw_ref[...]


def _prep(dega, degb, xw):
    g = pl.cdiv(N_PAD, BM)
    return pl.pallas_call(
        _prep_kernel,
        grid=(g,),
        in_specs=[
            pl.BlockSpec((BM, 128), lambda i: (i, 0)),
            pl.BlockSpec((BM, 128), lambda i: (i, 0)),
            pl.BlockSpec((BM, H), lambda i: (i, 0)),
        ],
        out_specs=[
            pl.BlockSpec((BM, 1), lambda i: (i, 0)),
            pl.BlockSpec((BM, H), lambda i: (i, 0)),
        ],
        out_shape=[
            jax.ShapeDtypeStruct((N_PAD, 1), jnp.float32),
            jax.ShapeDtypeStruct((N_PAD, H), jnp.float32),
        ],
    )(dega, degb, xw)


def _fused_layer_kernel(s_ref, dinv_ref, b_ref, w_ref, o_ref):
    h = jnp.maximum(dinv_ref[...] * (s_ref[0] + s_ref[1]) + b_ref[...], 0.0)
    o_ref[...] = dinv_ref[...] * jnp.dot(h, w_ref[...],
                                         preferred_element_type=jnp.float32)


def _fused_layer(s, dinv, b, w):
    g = pl.cdiv(N_PAD, BM)
    win = s.shape[2]
    return pl.pallas_call(
        _fused_layer_kernel,
        grid=(g,),
        in_specs=[
            pl.BlockSpec((2, BM, win), lambda i: (0, i, 0)),
            pl.BlockSpec((BM, 1), lambda i: (i, 0)),
            pl.BlockSpec((1, win), lambda i: (0, 0)),
            pl.BlockSpec(w.shape, lambda i: (0, 0)),
        ],
        out_specs=pl.BlockSpec((BM, w.shape[1]), lambda i: (i, 0)),
        out_shape=jax.ShapeDtypeStruct((N_PAD, w.shape[1]), jnp.float32),
    )(s, dinv, b, w)


def _mu_kernel(s_ref, dinv_ref, b_ref, o_ref):
    o_ref[...] = dinv_ref[...] * (s_ref[0][:, :Z] + s_ref[1][:, :Z]) + b_ref[...]


def _mu_combine(s, dinv, b):
    g = pl.cdiv(N_PAD, BM)
    return pl.pallas_call(
        _mu_kernel,
        grid=(g,),
        in_specs=[
            pl.BlockSpec((2, BM, 128), lambda i: (0, i, 0)),
            pl.BlockSpec((BM, 1), lambda i: (i, 0)),
            pl.BlockSpec((1, Z), lambda i: (0, 0)),
        ],
        out_specs=pl.BlockSpec((BM, Z), lambda i: (i, 0)),
        out_shape=jax.ShapeDtypeStruct((N_PAD, Z), jnp.float32),
    )(s, dinv, b)


BDM = 1024
BN = 2048


def _decoder_kernel(a_ref, b_ref, o_ref):
    p = lax.dot_general(a_ref[...], b_ref[...], (((1,), (1,)), ((), ())),
                        preferred_element_type=jnp.float32)
    o_ref[...] = 0.5 * jnp.tanh(0.5 * p) + 0.5


def _decoder(z):
    return pl.pallas_call(
        _decoder_kernel,
        grid=(pl.cdiv(N, BDM), pl.cdiv(N, BN)),
        in_specs=[
            pl.BlockSpec((BDM, Z), lambda i, j: (i, 0)),
            pl.BlockSpec((BN, Z), lambda i, j: (j, 0)),
        ],
        out_specs=pl.BlockSpec((BDM, BN), lambda i, j: (i, j)),
        out_shape=jax.ShapeDtypeStruct((N, N), jnp.float32),
    )(z, z)


def kernel(x, edge_index, W1, b1, W2, b2, Wmu, bmu, Wls, bls):
    del Wls, bls
    src = edge_index[0].astype(jnp.int32)
    dst = edge_index[1].astype(jnp.int32)

    npad = E_PAD - E
    pad_idx = (N + (jnp.arange(npad, dtype=jnp.int32) % 16))
    src_p = jnp.concatenate([src, pad_idx]).reshape(NW, NCHUNK, K)
    dst_p = jnp.concatenate([dst, pad_idx]).reshape(NW, NCHUNK, K)

    x_pad = jnp.pad(x, ((0, N_PAD - N), (0, 0)))
    zeros128 = jnp.zeros((N_PAD, 128), jnp.float32)
    ones128 = jnp.ones((K, 128), jnp.float32)
    Wmu_p = jnp.pad(Wmu, ((0, 0), (0, 128 - Z)))

    deg = _sc_deg(ones128, zeros128, dst_p)
    xw1 = _xw(x_pad, W1)
    dinv, u1 = _prep(deg[0], deg[1], xw1)

    s1 = _sc_scatter_128(u1, zeros128, src_p, dst_p)
    u2 = _fused_layer(s1, dinv, b1[None, :], W2)
    s2 = _sc_scatter_128(u2, zeros128, src_p, dst_p)
    u3 = _fused_layer(s2, dinv, b2[None, :], Wmu_p)
    s3 = _sc_scatter_128(u3, zeros128, src_p, dst_p)
    mu = _mu_combine(s3, dinv, bmu[None, :])
    return _decoder(mu)
